# baseline (speedup 1.0000x reference)
import functools

import jax
import jax.numpy as jnp
from jax import lax
from jax.experimental import pallas as pl
from jax.experimental.pallas import tpu as pltpu
from jax.experimental.pallas import tpu_sc as plsc


_GATHER_CHUNK = 80


@functools.lru_cache(maxsize=None)
def _make_gather(vocab, dim, b_padded):
    info = plsc.get_sparse_core_info()
    nc, ns = info.num_cores, info.num_subcores
    nw = nc * ns
    b_per_w = b_padded // nw
    n_chunks = b_per_w // _GATHER_CHUNK
    mesh = plsc.VectorSubcoreMesh(core_axis_name="c", subcore_axis_name="s")

    @functools.partial(
        pl.kernel,
        mesh=mesh,
        out_type=jax.ShapeDtypeStruct((b_padded, dim), jnp.float32),
        scratch_types=[
            pltpu.VMEM((n_chunks, _GATHER_CHUNK), jnp.int32),
            pltpu.VMEM((b_per_w, dim), jnp.float32),
            pltpu.SemaphoreType.DMA,
        ],
    )
    def gather(table_hbm, idx_hbm, out_hbm, idx_v, rows_v, sem):
        wid = lax.axis_index("s") * nc + lax.axis_index("c")
        base = wid * b_per_w
        for j in range(n_chunks):
            pltpu.sync_copy(
                idx_hbm.at[pl.ds(base + j * _GATHER_CHUNK, _GATHER_CHUNK)],
                idx_v.at[j],
            )
        copies = []
        for j in range(n_chunks):
            copies.append(
                pltpu.async_copy(
                    table_hbm.at[idx_v.at[j]],
                    rows_v.at[pl.ds(j * _GATHER_CHUNK, _GATHER_CHUNK)],
                    sem,
                )
            )
        for c in copies:
            c.wait()
        pltpu.sync_copy(rows_v, out_hbm.at[pl.ds(base, b_per_w)])

    return gather


def _layer0_body(x0_ref, adj_ref, w_ref, b_ref, x1_ref, adj16_ref, inv_ref):
    a = adj_ref[...]
    deg = jnp.sum(a, axis=1, keepdims=True)
    iv = 1.0 / (deg + 1e-6)
    inv_ref[...] = iv
    a16 = a.astype(jnp.bfloat16)
    adj16_ref[...] = a16
    x = x0_ref[...]
    agg = jnp.dot(a16, x.astype(jnp.bfloat16), preferred_element_type=jnp.float32)
    agg = agg * iv
    lin = jnp.dot(agg, w_ref[0], preferred_element_type=jnp.float32) + b_ref[0]
    blk = adj_ref.shape[0]
    r = pl.program_id(0)
    x1_ref[...] = jnp.maximum(lin, 0.0) + x0_ref[pl.ds(r * blk, blk), :]


def _layer0(x0, adj, w, b, blk):
    n, dim = x0.shape
    layers = w.shape[0]
    nb = n // blk
    return pl.pallas_call(
        _layer0_body,
        grid=(nb,),
        in_specs=[
            pl.BlockSpec((n, dim), lambda r: (0, 0)),
            pl.BlockSpec((blk, n), lambda r: (r, 0)),
            pl.BlockSpec((layers, dim, dim), lambda r: (0, 0, 0)),
            pl.BlockSpec((layers, dim), lambda r: (0, 0)),
        ],
        out_specs=[
            pl.BlockSpec((blk, dim), lambda r: (r, 0)),
            pl.BlockSpec((blk, n), lambda r: (r, 0)),
            pl.BlockSpec((blk, 1), lambda r: (r, 0)),
        ],
        out_shape=[
            jax.ShapeDtypeStruct((n, dim), jnp.float32),
            jax.ShapeDtypeStruct((n, n), jnp.bfloat16),
            jax.ShapeDtypeStruct((n, 1), jnp.float32),
        ],
        compiler_params=pltpu.CompilerParams(
            dimension_semantics=("arbitrary",),
        ),
    )(x0, adj, w, b)


def _rest_body(x1_ref, adj16_ref, inv_ref, w_ref, b_ref, out_ref, xa, xb):
    li = pl.program_id(0)
    r = pl.program_id(1)
    blk = adj16_ref.shape[0]
    a16 = adj16_ref[...]
    iv = inv_ref[pl.ds(r * blk, blk), :]
    wl = w_ref[li + 1]
    bl = b_ref[li + 1]

    def step(src_ref):
        x = src_ref[...]
        agg = jnp.dot(
            a16, x.astype(jnp.bfloat16), preferred_element_type=jnp.float32
        ) * iv
        lin = jnp.dot(agg, wl, preferred_element_type=jnp.float32) + bl
        return jnp.maximum(lin, 0.0) + src_ref[pl.ds(r * blk, blk), :]

    @pl.when(li == 0)
    def _():
        xa[pl.ds(r * blk, blk), :] = step(x1_ref)

    @pl.when(li == 1)
    def _():
        xb[pl.ds(r * blk, blk), :] = step(xa)

    @pl.when(li == 2)
    def _():
        h = step(xb)
        m = jnp.max(h, axis=0, keepdims=True)

        @pl.when(r == 0)
        def _():
            out_ref[...] = m

        @pl.when(r > 0)
        def _():
            out_ref[...] = jnp.maximum(out_ref[...], m)


def _rest(x1, adj16, inv, w, b, blk):
    n, dim = x1.shape
    layers = w.shape[0]
    nb = n // blk
    return pl.pallas_call(
        _rest_body,
        grid=(layers - 1, nb),
        in_specs=[
            pl.BlockSpec((n, dim), lambda l, r: (0, 0)),
            pl.BlockSpec((blk, n), lambda l, r: (r, 0)),
            pl.BlockSpec((n, 1), lambda l, r: (0, 0)),
            pl.BlockSpec((layers, dim, dim), lambda l, r: (0, 0, 0)),
            pl.BlockSpec((layers, dim), lambda l, r: (0, 0)),
        ],
        out_specs=pl.BlockSpec((1, dim), lambda l, r: (0, 0)),
        out_shape=jax.ShapeDtypeStruct((1, dim), jnp.float32),
        scratch_shapes=[
            pltpu.VMEM((n, dim), jnp.float32),
            pltpu.VMEM((n, dim), jnp.float32),
        ],
        compiler_params=pltpu.CompilerParams(
            dimension_semantics=("arbitrary", "arbitrary"),
        ),
    )(x1, adj16, inv, w, b)


def kernel(token_ids, adj, emb, W, b):
    n = adj.shape[0]
    vocab, dim = emb.shape

    info = plsc.get_sparse_core_info()
    nw = info.num_cores * info.num_subcores
    quantum = nw * _GATHER_CHUNK
    b_padded = ((n + quantum - 1) // quantum) * quantum
    ids = jnp.pad(token_ids.astype(jnp.int32), (0, b_padded - n))
    x0 = _make_gather(vocab, dim, b_padded)(emb, ids)[:n]

    x1, adj16, inv = _layer0(x0, adj, W, b, blk=400)
    pooled = _rest(x1, adj16, inv, W, b, blk=400)
    return pooled.reshape(dim)


# speedup vs baseline: 1.0021x; 1.0021x over previous
import functools

import jax
import jax.numpy as jnp
from jax import lax
from jax.experimental import pallas as pl
from jax.experimental.pallas import tpu as pltpu
from jax.experimental.pallas import tpu_sc as plsc


_GATHER_CHUNK = 80


@functools.lru_cache(maxsize=None)
def _make_gather(vocab, dim, b_padded):
    info = plsc.get_sparse_core_info()
    nc, ns = info.num_cores, info.num_subcores
    nw = nc * ns
    b_per_w = b_padded // nw
    n_chunks = b_per_w // _GATHER_CHUNK
    mesh = plsc.VectorSubcoreMesh(core_axis_name="c", subcore_axis_name="s")

    @functools.partial(
        pl.kernel,
        mesh=mesh,
        out_type=jax.ShapeDtypeStruct((b_padded, dim), jnp.float32),
        scratch_types=[
            pltpu.VMEM((n_chunks, _GATHER_CHUNK), jnp.int32),
            pltpu.VMEM((b_per_w, dim), jnp.float32),
            pltpu.SemaphoreType.DMA,
        ],
    )
    def gather(table_hbm, idx_hbm, out_hbm, idx_v, rows_v, sem):
        wid = lax.axis_index("s") * nc + lax.axis_index("c")
        base = wid * b_per_w
        for j in range(n_chunks):
            pltpu.sync_copy(
                idx_hbm.at[pl.ds(base + j * _GATHER_CHUNK, _GATHER_CHUNK)],
                idx_v.at[j],
            )
        copies = []
        for j in range(n_chunks):
            copies.append(
                pltpu.async_copy(
                    table_hbm.at[idx_v.at[j]],
                    rows_v.at[pl.ds(j * _GATHER_CHUNK, _GATHER_CHUNK)],
                    sem,
                )
            )
        for c in copies:
            c.wait()
        pltpu.sync_copy(rows_v, out_hbm.at[pl.ds(base, b_per_w)])

    return gather


def _layer0_body(x0_ref, adj_ref, w_ref, b_ref, x1_ref, adj16_ref):
    a = adj_ref[...]
    deg = jnp.sum(a, axis=1, keepdims=True)
    iv = 1.0 / (deg + 1e-6)
    a16 = (a * iv).astype(jnp.bfloat16)
    adj16_ref[...] = a16
    x = x0_ref[...]
    agg = jnp.dot(a16, x.astype(jnp.bfloat16), preferred_element_type=jnp.float32)
    lin = jnp.dot(agg, w_ref[0], preferred_element_type=jnp.float32) + b_ref[0]
    blk = adj_ref.shape[0]
    r = pl.program_id(0)
    x1_ref[...] = jnp.maximum(lin, 0.0) + x0_ref[pl.ds(r * blk, blk), :]


def _layer0(x0, adj, w, b, blk):
    n, dim = x0.shape
    layers = w.shape[0]
    nb = n // blk
    return pl.pallas_call(
        _layer0_body,
        grid=(nb,),
        in_specs=[
            pl.BlockSpec((n, dim), lambda r: (0, 0)),
            pl.BlockSpec((blk, n), lambda r: (r, 0)),
            pl.BlockSpec((layers, dim, dim), lambda r: (0, 0, 0)),
            pl.BlockSpec((layers, dim), lambda r: (0, 0)),
        ],
        out_specs=[
            pl.BlockSpec((blk, dim), lambda r: (r, 0)),
            pl.BlockSpec((blk, n), lambda r: (r, 0)),
        ],
        out_shape=[
            jax.ShapeDtypeStruct((n, dim), jnp.float32),
            jax.ShapeDtypeStruct((n, n), jnp.bfloat16),
        ],
        compiler_params=pltpu.CompilerParams(
            dimension_semantics=("arbitrary",),
        ),
    )(x0, adj, w, b)


def _rest_body(x1_ref, adj16_ref, w_ref, b_ref, out_ref, xa, xb):
    li = pl.program_id(0)
    r = pl.program_id(1)
    blk = adj16_ref.shape[0]
    a16 = adj16_ref[...]
    wl = w_ref[li + 1]
    bl = b_ref[li + 1]

    def step(src_ref):
        x = src_ref[...]
        agg = jnp.dot(
            a16, x.astype(jnp.bfloat16), preferred_element_type=jnp.float32
        )
        lin = jnp.dot(agg, wl, preferred_element_type=jnp.float32) + bl
        return jnp.maximum(lin, 0.0) + src_ref[pl.ds(r * blk, blk), :]

    @pl.when(li == 0)
    def _():
        xa[pl.ds(r * blk, blk), :] = step(x1_ref)

    @pl.when(li == 1)
    def _():
        xb[pl.ds(r * blk, blk), :] = step(xa)

    @pl.when(li == 2)
    def _():
        h = step(xb)
        m = jnp.max(h, axis=0, keepdims=True)

        @pl.when(r == 0)
        def _():
            out_ref[...] = m

        @pl.when(r > 0)
        def _():
            out_ref[...] = jnp.maximum(out_ref[...], m)


def _rest(x1, adj16, w, b, blk):
    n, dim = x1.shape
    layers = w.shape[0]
    nb = n // blk
    return pl.pallas_call(
        _rest_body,
        grid=(layers - 1, nb),
        in_specs=[
            pl.BlockSpec((n, dim), lambda l, r: (0, 0)),
            pl.BlockSpec((blk, n), lambda l, r: (r, 0)),
            pl.BlockSpec((layers, dim, dim), lambda l, r: (0, 0, 0)),
            pl.BlockSpec((layers, dim), lambda l, r: (0, 0)),
        ],
        out_specs=pl.BlockSpec((1, dim), lambda l, r: (0, 0)),
        out_shape=jax.ShapeDtypeStruct((1, dim), jnp.float32),
        scratch_shapes=[
            pltpu.VMEM((n, dim), jnp.float32),
            pltpu.VMEM((n, dim), jnp.float32),
        ],
        compiler_params=pltpu.CompilerParams(
            dimension_semantics=("arbitrary", "arbitrary"),
        ),
    )(x1, adj16, w, b)


def kernel(token_ids, adj, emb, W, b):
    n = adj.shape[0]
    vocab, dim = emb.shape

    info = plsc.get_sparse_core_info()
    nw = info.num_cores * info.num_subcores
    quantum = nw * _GATHER_CHUNK
    b_padded = ((n + quantum - 1) // quantum) * quantum
    ids = jnp.pad(token_ids.astype(jnp.int32), (0, b_padded - n))
    x0 = _make_gather(vocab, dim, b_padded)(emb, ids)[:n]

    x1, adj16 = _layer0(x0, adj, W, b, blk=200)
    pooled = _rest(x1, adj16, W, b, blk=400)
    return pooled.reshape(dim)


# speedup vs baseline: 1.0080x; 1.0059x over previous
import functools

import jax
import jax.numpy as jnp
from jax import lax
from jax.experimental import pallas as pl
from jax.experimental.pallas import tpu as pltpu
from jax.experimental.pallas import tpu_sc as plsc


_GATHER_CHUNK = 80


@functools.lru_cache(maxsize=None)
def _make_gather(vocab, dim, b_padded):
    info = plsc.get_sparse_core_info()
    nc, ns = info.num_cores, info.num_subcores
    nw = nc * ns
    b_per_w = b_padded // nw
    n_chunks = b_per_w // _GATHER_CHUNK
    mesh = plsc.VectorSubcoreMesh(core_axis_name="c", subcore_axis_name="s")

    @functools.partial(
        pl.kernel,
        mesh=mesh,
        out_type=jax.ShapeDtypeStruct((b_padded, dim), jnp.float32),
        scratch_types=[
            pltpu.VMEM((n_chunks, _GATHER_CHUNK), jnp.int32),
            pltpu.VMEM((b_per_w, dim), jnp.float32),
            pltpu.SemaphoreType.DMA,
        ],
    )
    def gather(table_hbm, idx_hbm, out_hbm, idx_v, rows_v, sem):
        wid = lax.axis_index("s") * nc + lax.axis_index("c")
        base = wid * b_per_w
        for j in range(n_chunks):
            pltpu.sync_copy(
                idx_hbm.at[pl.ds(base + j * _GATHER_CHUNK, _GATHER_CHUNK)],
                idx_v.at[j],
            )
        copies = []
        for j in range(n_chunks):
            copies.append(
                pltpu.async_copy(
                    table_hbm.at[idx_v.at[j]],
                    rows_v.at[pl.ds(j * _GATHER_CHUNK, _GATHER_CHUNK)],
                    sem,
                )
            )
        for c in copies:
            c.wait()
        pltpu.sync_copy(rows_v, out_hbm.at[pl.ds(base, b_per_w)])

    return gather


def _layer0_body(x0_ref, adj_ref, w_ref, b_ref, x1_ref, adj16_ref):
    a = adj_ref[...]
    deg = jnp.sum(a, axis=1, keepdims=True)
    iv = 1.0 / (deg + 1e-6)
    a16 = (a * iv).astype(jnp.bfloat16)
    adj16_ref[...] = a16
    x = x0_ref[...]
    agg = jnp.dot(a16, x.astype(jnp.bfloat16), preferred_element_type=jnp.float32)
    lin = jnp.dot(agg, w_ref[0], preferred_element_type=jnp.float32) + b_ref[0]
    blk = adj_ref.shape[0]
    r = pl.program_id(0)
    x1_ref[...] = jnp.maximum(lin, 0.0) + x0_ref[pl.ds(r * blk, blk), :]


def _layer0(x0, adj, w, b, blk):
    n, dim = x0.shape
    layers = w.shape[0]
    nb = n // blk
    return pl.pallas_call(
        _layer0_body,
        grid=(nb,),
        in_specs=[
            pl.BlockSpec((n, dim), lambda r: (0, 0)),
            pl.BlockSpec((blk, n), lambda r: (r, 0)),
            pl.BlockSpec((1, dim, dim), lambda r: (0, 0, 0)),
            pl.BlockSpec((layers, dim), lambda r: (0, 0)),
        ],
        out_specs=[
            pl.BlockSpec((blk, dim), lambda r: (r, 0)),
            pl.BlockSpec((blk, n), lambda r: (r, 0)),
        ],
        out_shape=[
            jax.ShapeDtypeStruct((n, dim), jnp.float32),
            jax.ShapeDtypeStruct((n, n), jnp.bfloat16),
        ],
        compiler_params=pltpu.CompilerParams(
            dimension_semantics=("arbitrary",),
        ),
    )(x0, adj, w, b)


def _rest_body(x1_ref, adj16_ref, w_ref, b_ref, out_ref, xa, xb):
    li = pl.program_id(0)
    r = pl.program_id(1)
    blk = adj16_ref.shape[0]
    a16 = adj16_ref[...]
    wl = w_ref[li + 1]
    bl = b_ref[li + 1]

    def step(src_ref):
        x = src_ref[...]
        agg = jnp.dot(
            a16, x.astype(jnp.bfloat16), preferred_element_type=jnp.float32
        )
        lin = jnp.dot(agg, wl, preferred_element_type=jnp.float32) + bl
        return jnp.maximum(lin, 0.0) + src_ref[pl.ds(r * blk, blk), :]

    @pl.when(li == 0)
    def _():
        xa[pl.ds(r * blk, blk), :] = step(x1_ref)

    @pl.when(li == 1)
    def _():
        xb[pl.ds(r * blk, blk), :] = step(xa)

    @pl.when(li == 2)
    def _():
        h = step(xb)
        m = jnp.max(h, axis=0, keepdims=True)

        @pl.when(r == 0)
        def _():
            out_ref[...] = m

        @pl.when(r > 0)
        def _():
            out_ref[...] = jnp.maximum(out_ref[...], m)


def _rest(x1, adj16, w, b, blk):
    n, dim = x1.shape
    layers = w.shape[0]
    nb = n // blk
    return pl.pallas_call(
        _rest_body,
        grid=(layers - 1, nb),
        in_specs=[
            pl.BlockSpec((n, dim), lambda l, r: (0, 0)),
            pl.BlockSpec((blk, n), lambda l, r: (r, 0)),
            pl.BlockSpec((layers, dim, dim), lambda l, r: (0, 0, 0)),
            pl.BlockSpec((layers, dim), lambda l, r: (0, 0)),
        ],
        out_specs=pl.BlockSpec((1, dim), lambda l, r: (0, 0)),
        out_shape=jax.ShapeDtypeStruct((1, dim), jnp.float32),
        scratch_shapes=[
            pltpu.VMEM((n, dim), jnp.float32),
            pltpu.VMEM((n, dim), jnp.float32),
        ],
        compiler_params=pltpu.CompilerParams(
            dimension_semantics=("arbitrary", "arbitrary"),
        ),
    )(x1, adj16, w, b)


def kernel(token_ids, adj, emb, W, b):
    n = adj.shape[0]
    vocab, dim = emb.shape

    info = plsc.get_sparse_core_info()
    nw = info.num_cores * info.num_subcores
    quantum = nw * _GATHER_CHUNK
    b_padded = ((n + quantum - 1) // quantum) * quantum
    ids = jnp.pad(token_ids.astype(jnp.int32), (0, b_padded - n))
    x0 = _make_gather(vocab, dim, b_padded)(emb, ids)[:n]

    x1, adj16 = _layer0(x0, adj, W, b, blk=400)
    pooled = _rest(x1, adj16, W, b, blk=400)
    return pooled.reshape(dim)
